# Initial kernel scaffold; baseline (speedup 1.0000x reference)
import jax
import jax.numpy as jnp
from jax.experimental import pallas as pl


def kernel(x, edge_index, batch, W1l, b1, W1r, W2l, b2, W2r, Wo, bo):
    raise NotImplementedError("write your pallas kernel here")


# speedup vs baseline: 3.1189x; 3.1189x over previous
import functools

import jax
import jax.numpy as jnp
from jax import lax
from jax.experimental import pallas as pl
from jax.experimental.pallas import tpu as pltpu
from jax.experimental.pallas import tpu_sc as plsc

N = 10000
D = 128
G = 16
NC, NS = 2, 16
NW = NC * NS
CH = 128
NPAD = 10112
ZR = NPAD // NS
W0 = 632
WLAST = N - (NS - 1) * W0
R = 1000

@functools.cache
def _mesh():
    return plsc.VectorSubcoreMesh(core_axis_name="c", subcore_axis_name="s")


def _writeback(sh_ref, out_hbm, cid, sid):
    @pl.when(sid < NS - 1)
    def _():
        pltpu.sync_copy(sh_ref.at[pl.ds(sid * W0, W0)],
                        out_hbm.at[pl.ds(cid * N + sid * W0, W0)])

    @pl.when(sid == NS - 1)
    def _():
        base = (NS - 1) * W0
        pltpu.sync_copy(sh_ref.at[pl.ds(base, WLAST)],
                        out_hbm.at[pl.ds(cid * N + base, WLAST)])


def _sc_agg_body(K, x_hbm, src_hbm, dst_hbm, zacc_hbm, accout_hbm,
                 src_v, dst_v, rows_v, acc_sh):
    cid = lax.axis_index("c")
    sid = lax.axis_index("s")
    wid = cid * NS + sid

    pltpu.sync_copy(zacc_hbm.at[pl.ds(sid * ZR, ZR)],
                    acc_sh.at[pl.ds(sid * ZR, ZR)])
    plsc.subcore_barrier()

    @pl.loop(0, K // 8)
    def _(b):
        pltpu.sync_copy(src_hbm.at[wid, pl.ds(b * 8, 8)], src_v)
        pltpu.sync_copy(dst_hbm.at[wid, pl.ds(b * 8, 8)], dst_v)

        @pl.loop(0, 8)
        def _(j):
            pltpu.sync_copy(x_hbm.at[src_v.at[j]], rows_v)
            pltpu.sync_copy(rows_v, acc_sh.at[dst_v.at[j]], add=True)

    plsc.subcore_barrier()
    _writeback(acc_sh, accout_hbm, cid, sid)


def _sc_agg(x, src3, dst3):
    K = src3.shape[1]
    scratch = [
        pltpu.VMEM((8, CH), jnp.int32),
        pltpu.VMEM((8, CH), jnp.int32),
        pltpu.VMEM((CH, D), jnp.float32),
        pltpu.VMEM_SHARED((NPAD, D), jnp.float32),
    ]
    k = pl.kernel(functools.partial(_sc_agg_body, K),
                  out_type=jax.ShapeDtypeStruct((NC * N, D), jnp.float32),
                  mesh=_mesh(), scratch_types=scratch)
    return k(x, src3, dst3, jnp.zeros((NPAD, D), jnp.float32))


CW = 128


def _sc_cnt_body(K, dst_hbm, zcnt_hbm, ones_hbm, cntout_hbm,
                 dst_v, ones_v, cnt_sh):
    cid = lax.axis_index("c")
    sid = lax.axis_index("s")
    wid = cid * NS + sid

    pltpu.sync_copy(zcnt_hbm.at[pl.ds(sid * ZR, ZR)],
                    cnt_sh.at[pl.ds(sid * ZR, ZR)])
    pltpu.sync_copy(ones_hbm, ones_v)
    plsc.subcore_barrier()

    @pl.loop(0, K // 8)
    def _(b):
        pltpu.sync_copy(dst_hbm.at[wid, pl.ds(b * 8, 8)], dst_v)

        @pl.loop(0, 8)
        def _(j):
            pltpu.sync_copy(ones_v, cnt_sh.at[dst_v.at[j]], add=True)

    plsc.subcore_barrier()
    _writeback(cnt_sh, cntout_hbm, cid, sid)


def _sc_cnt(dst3):
    K = dst3.shape[1]
    scratch = [
        pltpu.VMEM((8, CH), jnp.int32),
        pltpu.VMEM((CH, CW), jnp.float32),
        pltpu.VMEM_SHARED((NPAD, CW), jnp.float32),
    ]
    k = pl.kernel(functools.partial(_sc_cnt_body, K),
                  out_type=jax.ShapeDtypeStruct((NC * N, CW), jnp.float32),
                  mesh=_mesh(), scratch_types=scratch)
    return k(dst3, jnp.zeros((NPAD, CW), jnp.float32),
             jnp.ones((CH, CW), jnp.float32))


def _xw_body(x_ref, w_ref, b_ref, o_ref):
    o_ref[...] = (jnp.dot(x_ref[...], w_ref[...],
                          preferred_element_type=jnp.float32) + b_ref[...])


def _xw_b(x, W, b2d):
    return pl.pallas_call(
        _xw_body,
        grid=(N // R,),
        in_specs=[pl.BlockSpec((R, D), lambda i: (i, 0)),
                  pl.BlockSpec((D, D), lambda i: (0, 0)),
                  pl.BlockSpec((1, D), lambda i: (0, 0))],
        out_specs=pl.BlockSpec((R, D), lambda i: (i, 0)),
        out_shape=jax.ShapeDtypeStruct((N, D), jnp.float32),
    )(x, W, b2d)


def _mean_of(p0, p1, c0, c1):
    cnt = jnp.maximum((c0 + c1)[:, 0:1], 1.0)
    return (p0 + p1) / cnt


def _comb_body(p0_ref, p1_ref, c0_ref, c1_ref, xr_ref, w_ref, o_ref):
    mean = _mean_of(p0_ref[...], p1_ref[...], c0_ref[...], c1_ref[...])
    o_ref[...] = jnp.maximum(
        jnp.dot(mean, w_ref[...], preferred_element_type=jnp.float32)
        + xr_ref[...], 0.0)


def _layer_combine(parts, cnts, xr, Wl):
    nb = N // R
    return pl.pallas_call(
        _comb_body,
        grid=(nb,),
        in_specs=[pl.BlockSpec((R, D), lambda i: (i, 0)),
                  pl.BlockSpec((R, D), lambda i: (i + nb, 0)),
                  pl.BlockSpec((R, CW), lambda i: (i, 0)),
                  pl.BlockSpec((R, CW), lambda i: (i + nb, 0)),
                  pl.BlockSpec((R, D), lambda i: (i, 0)),
                  pl.BlockSpec((D, D), lambda i: (0, 0))],
        out_specs=pl.BlockSpec((R, D), lambda i: (i, 0)),
        out_shape=jax.ShapeDtypeStruct((N, D), jnp.float32),
    )(parts, parts, cnts, cnts, xr, Wl)


def _final_body(p0_ref, p1_ref, c0_ref, c1_ref, xr_ref, w_ref, b_ref,
                wo_ref, bo_ref, o_ref, gsum, gcnt):
    i = pl.program_id(0)

    @pl.when(i == 0)
    def _():
        gsum[...] = jnp.zeros_like(gsum)
        gcnt[...] = jnp.zeros_like(gcnt)

    mean = _mean_of(p0_ref[...], p1_ref[...], c0_ref[...], c1_ref[...])
    h = jnp.maximum(
        jnp.dot(mean, w_ref[...], preferred_element_type=jnp.float32)
        + xr_ref[...], 0.0)
    oh = (b_ref[...] == lax.broadcasted_iota(jnp.int32, (R, G), 1)
          ).astype(jnp.float32)
    gsum[...] += lax.dot_general(oh, h, (((0,), (0,)), ((), ())),
                                 preferred_element_type=jnp.float32)
    gcnt[...] += lax.dot_general(oh, jnp.ones((R, D), jnp.float32),
                                 (((0,), (0,)), ((), ())),
                                 preferred_element_type=jnp.float32)

    @pl.when(i == pl.num_programs(0) - 1)
    def _():
        g = gsum[...] / jnp.maximum(gcnt[...], 1.0)
        o_ref[...] = (jnp.dot(g, wo_ref[...],
                              preferred_element_type=jnp.float32) + bo_ref[...])


def _final(parts, cnts, xr, Wl, batch2d, Wo, bo2d):
    nb = N // R
    return pl.pallas_call(
        _final_body,
        grid=(nb,),
        in_specs=[pl.BlockSpec((R, D), lambda i: (i, 0)),
                  pl.BlockSpec((R, D), lambda i: (i + nb, 0)),
                  pl.BlockSpec((R, CW), lambda i: (i, 0)),
                  pl.BlockSpec((R, CW), lambda i: (i + nb, 0)),
                  pl.BlockSpec((R, D), lambda i: (i, 0)),
                  pl.BlockSpec((D, D), lambda i: (0, 0)),
                  pl.BlockSpec((R, 1), lambda i: (i, 0)),
                  pl.BlockSpec((D, 1), lambda i: (0, 0)),
                  pl.BlockSpec((1, 1), lambda i: (0, 0))],
        out_specs=pl.BlockSpec((G, 1), lambda i: (0, 0)),
        out_shape=jax.ShapeDtypeStruct((G, 1), jnp.float32),
        scratch_shapes=[pltpu.VMEM((G, D), jnp.float32),
                        pltpu.VMEM((G, D), jnp.float32)],
    )(parts, parts, cnts, cnts, xr, Wl, batch2d, Wo, bo2d)


def kernel(x, edge_index, batch, W1l, b1, W1r, W2l, b2, W2r, Wo, bo):
    E = edge_index.shape[1]
    epw = -(-E // (NW * CH * 8)) * CH * 8
    K = epw // CH
    pad = NW * epw - E
    src = jnp.concatenate([edge_index[0], jnp.zeros((pad,), jnp.int32)])
    dst = jnp.concatenate([edge_index[1], jnp.full((pad,), N, jnp.int32)])
    src3 = src.reshape(NW, K, CH)
    dst3 = dst.reshape(NW, K, CH)
    b1_2d = b1.reshape(1, D)
    b2_2d = b2.reshape(1, D)
    bo_2d = bo.reshape(1, 1)
    batch2d = batch.reshape(N, 1)

    cnts = _sc_cnt(dst3)
    parts1 = _sc_agg(x, src3, dst3)
    xr1 = _xw_b(x, W1r, b1_2d)
    h1 = _layer_combine(parts1, cnts, xr1, W1l)

    parts2 = _sc_agg(h1, src3, dst3)
    xr2 = _xw_b(h1, W2r, b2_2d)
    return _final(parts2, cnts, xr2, W2l, batch2d, Wo, bo_2d)


# speedup vs baseline: 3.4845x; 1.1172x over previous
import functools

import jax
import jax.numpy as jnp
from jax import lax
from jax.experimental import pallas as pl
from jax.experimental.pallas import tpu as pltpu
from jax.experimental.pallas import tpu_sc as plsc

N = 10000
D = 128
G = 16
NC, NS = 2, 16
NW = NC * NS
CH = 128
NPAD = 10112
ZR = NPAD // NS
W0 = 632
WLAST = N - (NS - 1) * W0
R = 1000

@functools.cache
def _mesh():
    return plsc.VectorSubcoreMesh(core_axis_name="c", subcore_axis_name="s")


def _writeback(sh_ref, out_hbm, cid, sid):
    @pl.when(sid < NS - 1)
    def _():
        pltpu.sync_copy(sh_ref.at[pl.ds(sid * W0, W0)],
                        out_hbm.at[pl.ds(cid * N + sid * W0, W0)])

    @pl.when(sid == NS - 1)
    def _():
        base = (NS - 1) * W0
        pltpu.sync_copy(sh_ref.at[pl.ds(base, WLAST)],
                        out_hbm.at[pl.ds(cid * N + base, WLAST)])


def _zero_acc(rows_a, acc_sh, sid):
    z = jnp.zeros((16,), jnp.float32)

    @pl.loop(0, CH)
    def _(i):
        @pl.loop(0, D // 16)
        def _(l):
            rows_a.at[i, pl.ds(l * 16, 16)][...] = z

    @pl.loop(0, ZR // CH)
    def _(m):
        pltpu.sync_copy(rows_a,
                        acc_sh.at[pl.ds(sid * ZR + m * CH, CH)])
    rem = ZR % CH
    if rem:
        pltpu.sync_copy(rows_a.at[pl.ds(0, rem)],
                        acc_sh.at[pl.ds(sid * ZR + (ZR // CH) * CH, rem)])


def _sc_agg_body(K, x_hbm, idx_hbm, accout_hbm,
                 idx_v, rows_a, rows_b, acc_sh, sem_a, sem_b):
    cid = lax.axis_index("c")
    sid = lax.axis_index("s")
    wid = cid * NS + sid

    _zero_acc(rows_a, acc_sh, sid)
    plsc.subcore_barrier()

    def gather(buf, sem, j):
        return pltpu.async_copy(x_hbm.at[idx_v.at[2 * j]], buf, sem)

    def scat(buf, j):
        pltpu.sync_copy(buf, acc_sh.at[idx_v.at[2 * j + 1]], add=True)

    def wait(buf, sem):
        pltpu.make_async_copy(x_hbm.at[idx_v.at[0]], buf, sem).wait()

    @pl.loop(0, K // 8)
    def _(b):
        pltpu.sync_copy(idx_hbm.at[wid, pl.ds(b * 16, 16)], idx_v)
        gather(rows_a, sem_a, 0)
        for jj in range(7):
            pa = jj % 2 == 0
            gather(rows_b if pa else rows_a,
                   sem_b if pa else sem_a, jj + 1)
            wait(rows_a if pa else rows_b, sem_a if pa else sem_b)
            scat(rows_a if pa else rows_b, jj)
        wait(rows_b, sem_b)
        scat(rows_b, 7)

    plsc.subcore_barrier()
    _writeback(acc_sh, accout_hbm, cid, sid)


def _sc_agg(x, idx3):
    K = idx3.shape[1] // 2
    scratch = [
        pltpu.VMEM((16, CH), jnp.int32),
        pltpu.VMEM((CH, D), jnp.float32),
        pltpu.VMEM((CH, D), jnp.float32),
        pltpu.VMEM_SHARED((NPAD, D), jnp.float32),
        pltpu.SemaphoreType.DMA,
        pltpu.SemaphoreType.DMA,
    ]
    k = pl.kernel(functools.partial(_sc_agg_body, K),
                  out_type=jax.ShapeDtypeStruct((NC * N, D), jnp.float32),
                  mesh=_mesh(), scratch_types=scratch)
    return k(x, idx3)


CW = 128


def _sc_cnt_body(K, dst_hbm, zcnt_hbm, ones_hbm, cntout_hbm,
                 dst_v, ones_v, cnt_sh):
    cid = lax.axis_index("c")
    sid = lax.axis_index("s")
    wid = cid * NS + sid

    pltpu.sync_copy(zcnt_hbm.at[pl.ds(sid * ZR, ZR)],
                    cnt_sh.at[pl.ds(sid * ZR, ZR)])
    pltpu.sync_copy(ones_hbm, ones_v)
    plsc.subcore_barrier()

    @pl.loop(0, K // 8)
    def _(b):
        pltpu.sync_copy(dst_hbm.at[wid, pl.ds(b * 8, 8)], dst_v)

        @pl.loop(0, 8)
        def _(j):
            pltpu.sync_copy(ones_v, cnt_sh.at[dst_v.at[j]], add=True)

    plsc.subcore_barrier()
    _writeback(cnt_sh, cntout_hbm, cid, sid)


def _sc_cnt(dst3):
    K = dst3.shape[1]
    scratch = [
        pltpu.VMEM((8, CH), jnp.int32),
        pltpu.VMEM((CH, CW), jnp.float32),
        pltpu.VMEM_SHARED((NPAD, CW), jnp.float32),
    ]
    k = pl.kernel(functools.partial(_sc_cnt_body, K),
                  out_type=jax.ShapeDtypeStruct((NC * N, CW), jnp.float32),
                  mesh=_mesh(), scratch_types=scratch)
    return k(dst3, jnp.zeros((NPAD, CW), jnp.float32),
             jnp.ones((CH, CW), jnp.float32))


def _xw_body(x_ref, w_ref, b_ref, o_ref):
    o_ref[...] = (jnp.dot(x_ref[...], w_ref[...],
                          preferred_element_type=jnp.float32) + b_ref[...])


def _xw_b(x, W, b2d):
    return pl.pallas_call(
        _xw_body,
        grid=(N // R,),
        in_specs=[pl.BlockSpec((R, D), lambda i: (i, 0)),
                  pl.BlockSpec((D, D), lambda i: (0, 0)),
                  pl.BlockSpec((1, D), lambda i: (0, 0))],
        out_specs=pl.BlockSpec((R, D), lambda i: (i, 0)),
        out_shape=jax.ShapeDtypeStruct((N, D), jnp.float32),
    )(x, W, b2d)


def _mean_of(p0, p1, c0, c1):
    cnt = jnp.maximum((c0 + c1)[:, 0:1], 1.0)
    return (p0 + p1) / cnt


def _comb_body(p0_ref, p1_ref, c0_ref, c1_ref, xr_ref, w_ref, o_ref):
    mean = _mean_of(p0_ref[...], p1_ref[...], c0_ref[...], c1_ref[...])
    o_ref[...] = jnp.maximum(
        jnp.dot(mean, w_ref[...], preferred_element_type=jnp.float32)
        + xr_ref[...], 0.0)


def _layer_combine(parts, cnts, xr, Wl):
    nb = N // R
    return pl.pallas_call(
        _comb_body,
        grid=(nb,),
        in_specs=[pl.BlockSpec((R, D), lambda i: (i, 0)),
                  pl.BlockSpec((R, D), lambda i: (i + nb, 0)),
                  pl.BlockSpec((R, CW), lambda i: (i, 0)),
                  pl.BlockSpec((R, CW), lambda i: (i + nb, 0)),
                  pl.BlockSpec((R, D), lambda i: (i, 0)),
                  pl.BlockSpec((D, D), lambda i: (0, 0))],
        out_specs=pl.BlockSpec((R, D), lambda i: (i, 0)),
        out_shape=jax.ShapeDtypeStruct((N, D), jnp.float32),
    )(parts, parts, cnts, cnts, xr, Wl)


def _final_body(p0_ref, p1_ref, c0_ref, c1_ref, xr_ref, w_ref, b_ref,
                wo_ref, bo_ref, o_ref, gsum, gcnt):
    i = pl.program_id(0)

    @pl.when(i == 0)
    def _():
        gsum[...] = jnp.zeros_like(gsum)
        gcnt[...] = jnp.zeros_like(gcnt)

    mean = _mean_of(p0_ref[...], p1_ref[...], c0_ref[...], c1_ref[...])
    h = jnp.maximum(
        jnp.dot(mean, w_ref[...], preferred_element_type=jnp.float32)
        + xr_ref[...], 0.0)
    oh = (b_ref[...] == lax.broadcasted_iota(jnp.int32, (R, G), 1)
          ).astype(jnp.float32)
    gsum[...] += lax.dot_general(oh, h, (((0,), (0,)), ((), ())),
                                 preferred_element_type=jnp.float32)
    gcnt[...] += lax.dot_general(oh, jnp.ones((R, D), jnp.float32),
                                 (((0,), (0,)), ((), ())),
                                 preferred_element_type=jnp.float32)

    @pl.when(i == pl.num_programs(0) - 1)
    def _():
        g = gsum[...] / jnp.maximum(gcnt[...], 1.0)
        o_ref[...] = (jnp.dot(g, wo_ref[...],
                              preferred_element_type=jnp.float32) + bo_ref[...])


def _final(parts, cnts, xr, Wl, batch2d, Wo, bo2d):
    nb = N // R
    return pl.pallas_call(
        _final_body,
        grid=(nb,),
        in_specs=[pl.BlockSpec((R, D), lambda i: (i, 0)),
                  pl.BlockSpec((R, D), lambda i: (i + nb, 0)),
                  pl.BlockSpec((R, CW), lambda i: (i, 0)),
                  pl.BlockSpec((R, CW), lambda i: (i + nb, 0)),
                  pl.BlockSpec((R, D), lambda i: (i, 0)),
                  pl.BlockSpec((D, D), lambda i: (0, 0)),
                  pl.BlockSpec((R, 1), lambda i: (i, 0)),
                  pl.BlockSpec((D, 1), lambda i: (0, 0)),
                  pl.BlockSpec((1, 1), lambda i: (0, 0))],
        out_specs=pl.BlockSpec((G, 1), lambda i: (0, 0)),
        out_shape=jax.ShapeDtypeStruct((G, 1), jnp.float32),
        scratch_shapes=[pltpu.VMEM((G, D), jnp.float32),
                        pltpu.VMEM((G, D), jnp.float32)],
    )(parts, parts, cnts, cnts, xr, Wl, batch2d, Wo, bo2d)


def kernel(x, edge_index, batch, W1l, b1, W1r, W2l, b2, W2r, Wo, bo):
    E = edge_index.shape[1]
    epw = -(-E // (NW * CH * 8)) * CH * 8
    K = epw // CH
    pad = NW * epw - E
    src = jnp.concatenate([edge_index[0], jnp.zeros((pad,), jnp.int32)])
    dst = jnp.concatenate([edge_index[1], jnp.full((pad,), N, jnp.int32)])
    src3 = src.reshape(NW, K, CH)
    dst3 = dst.reshape(NW, K, CH)
    idx3 = jnp.stack([src3, dst3], axis=2).reshape(NW, 2 * K, CH)
    b1_2d = b1.reshape(1, D)
    b2_2d = b2.reshape(1, D)
    bo_2d = bo.reshape(1, 1)
    batch2d = batch.reshape(N, 1)

    cnts = _sc_cnt(dst3)
    parts1 = _sc_agg(x, idx3)
    xr1 = _xw_b(x, W1r, b1_2d)
    h1 = _layer_combine(parts1, cnts, xr1, W1l)

    parts2 = _sc_agg(h1, idx3)
    xr2 = _xw_b(h1, W2r, b2_2d)
    return _final(parts2, cnts, xr2, W2l, batch2d, Wo, bo_2d)


# speedup vs baseline: 3.4862x; 1.0005x over previous
import functools

import jax
import jax.numpy as jnp
from jax import lax
from jax.experimental import pallas as pl
from jax.experimental.pallas import tpu as pltpu
from jax.experimental.pallas import tpu_sc as plsc

N = 10000
D = 128
G = 16
NC, NS = 2, 16
NW = NC * NS
CH = 128
NPAD = 10112
ZR = NPAD // NS
W0 = 632
WLAST = N - (NS - 1) * W0
R = 1000

@functools.cache
def _mesh():
    return plsc.VectorSubcoreMesh(core_axis_name="c", subcore_axis_name="s")


def _writeback(sh_ref, out_hbm, cid, sid):
    @pl.when(sid < NS - 1)
    def _():
        pltpu.sync_copy(sh_ref.at[pl.ds(sid * W0, W0)],
                        out_hbm.at[pl.ds(cid * N + sid * W0, W0)])

    @pl.when(sid == NS - 1)
    def _():
        base = (NS - 1) * W0
        pltpu.sync_copy(sh_ref.at[pl.ds(base, WLAST)],
                        out_hbm.at[pl.ds(cid * N + base, WLAST)])


def _zero_acc(rows_a, acc_sh, sid):
    z = jnp.zeros((16,), jnp.float32)

    @pl.loop(0, CH)
    def _(i):
        @pl.loop(0, D // 16)
        def _(l):
            rows_a.at[i, pl.ds(l * 16, 16)][...] = z

    @pl.loop(0, ZR // CH)
    def _(m):
        pltpu.sync_copy(rows_a,
                        acc_sh.at[pl.ds(sid * ZR + m * CH, CH)])
    rem = ZR % CH
    if rem:
        pltpu.sync_copy(rows_a.at[pl.ds(0, rem)],
                        acc_sh.at[pl.ds(sid * ZR + (ZR // CH) * CH, rem)])


def _sc_agg_body(K, x_hbm, idx_hbm, accout_hbm,
                 idx_v, rows_a, rows_b, acc_sh,
                 gsem_a, gsem_b, ssem_a, ssem_b):
    cid = lax.axis_index("c")
    sid = lax.axis_index("s")
    wid = cid * NS + sid

    _zero_acc(rows_a, acc_sh, sid)
    plsc.subcore_barrier()

    def g(buf, sem, j):
        pltpu.async_copy(x_hbm.at[idx_v.at[2 * j]], buf, sem)

    def gwait(buf, sem):
        pltpu.make_async_copy(x_hbm.at[idx_v.at[0]], buf, sem).wait()

    def s(buf, sem, j):
        pltpu.async_copy(buf, acc_sh.at[idx_v.at[2 * j + 1]], sem, add=True)

    def swait(buf, sem):
        pltpu.make_async_copy(buf, acc_sh.at[idx_v.at[1]], sem).wait()

    A = (rows_a, gsem_a, ssem_a)
    B = (rows_b, gsem_b, ssem_b)

    @pl.loop(0, K // 8)
    def _(b):
        pltpu.sync_copy(idx_hbm.at[wid, pl.ds(b * 16, 16)], idx_v)
        g(rows_a, gsem_a, 0)
        for jj in range(1, 8):
            (bp, gp, sp), (bq, gq, sq) = (A, B) if jj % 2 == 0 else (B, A)
            if jj >= 2:
                swait(bp, sp)
            g(bp, gp, jj)
            gwait(bq, gq)
            s(bq, sq, jj - 1)
        gwait(rows_b, gsem_b)
        s(rows_b, ssem_b, 7)
        swait(rows_a, ssem_a)
        swait(rows_b, ssem_b)

    plsc.subcore_barrier()
    _writeback(acc_sh, accout_hbm, cid, sid)


def _sc_agg(x, idx3):
    K = idx3.shape[1] // 2
    scratch = [
        pltpu.VMEM((16, CH), jnp.int32),
        pltpu.VMEM((CH, D), jnp.float32),
        pltpu.VMEM((CH, D), jnp.float32),
        pltpu.VMEM_SHARED((NPAD, D), jnp.float32),
        pltpu.SemaphoreType.DMA,
        pltpu.SemaphoreType.DMA,
        pltpu.SemaphoreType.DMA,
        pltpu.SemaphoreType.DMA,
    ]
    k = pl.kernel(functools.partial(_sc_agg_body, K),
                  out_type=jax.ShapeDtypeStruct((NC * N, D), jnp.float32),
                  mesh=_mesh(), scratch_types=scratch)
    return k(x, idx3)


CW = 128


def _sc_cnt_body(K, dst_hbm, zcnt_hbm, ones_hbm, cntout_hbm,
                 dst_v, ones_v, cnt_sh, sem):
    cid = lax.axis_index("c")
    sid = lax.axis_index("s")
    wid = cid * NS + sid

    pltpu.sync_copy(zcnt_hbm.at[pl.ds(sid * ZR, ZR)],
                    cnt_sh.at[pl.ds(sid * ZR, ZR)])
    pltpu.sync_copy(ones_hbm, ones_v)
    plsc.subcore_barrier()

    @pl.loop(0, K // 8)
    def _(b):
        pltpu.sync_copy(dst_hbm.at[wid, pl.ds(b * 8, 8)], dst_v)
        for j in range(8):
            pltpu.async_copy(ones_v, cnt_sh.at[dst_v.at[j]], sem, add=True)
        for j in range(8):
            pltpu.make_async_copy(ones_v, cnt_sh.at[dst_v.at[0]], sem).wait()

    plsc.subcore_barrier()
    _writeback(cnt_sh, cntout_hbm, cid, sid)


def _sc_cnt(dst3):
    K = dst3.shape[1]
    scratch = [
        pltpu.VMEM((8, CH), jnp.int32),
        pltpu.VMEM((CH, CW), jnp.float32),
        pltpu.VMEM_SHARED((NPAD, CW), jnp.float32),
        pltpu.SemaphoreType.DMA,
    ]
    k = pl.kernel(functools.partial(_sc_cnt_body, K),
                  out_type=jax.ShapeDtypeStruct((NC * N, CW), jnp.float32),
                  mesh=_mesh(), scratch_types=scratch)
    return k(dst3, jnp.zeros((NPAD, CW), jnp.float32),
             jnp.ones((CH, CW), jnp.float32))


def _xw_body(x_ref, w_ref, b_ref, o_ref):
    o_ref[...] = (jnp.dot(x_ref[...], w_ref[...],
                          preferred_element_type=jnp.float32) + b_ref[...])


def _xw_b(x, W, b2d):
    return pl.pallas_call(
        _xw_body,
        grid=(N // R,),
        in_specs=[pl.BlockSpec((R, D), lambda i: (i, 0)),
                  pl.BlockSpec((D, D), lambda i: (0, 0)),
                  pl.BlockSpec((1, D), lambda i: (0, 0))],
        out_specs=pl.BlockSpec((R, D), lambda i: (i, 0)),
        out_shape=jax.ShapeDtypeStruct((N, D), jnp.float32),
    )(x, W, b2d)


def _mean_of(p0, p1, c0, c1):
    cnt = jnp.maximum((c0 + c1)[:, 0:1], 1.0)
    return (p0 + p1) / cnt


def _comb_body(p0_ref, p1_ref, c0_ref, c1_ref, xr_ref, w_ref, o_ref):
    mean = _mean_of(p0_ref[...], p1_ref[...], c0_ref[...], c1_ref[...])
    o_ref[...] = jnp.maximum(
        jnp.dot(mean, w_ref[...], preferred_element_type=jnp.float32)
        + xr_ref[...], 0.0)


def _layer_combine(parts, cnts, xr, Wl):
    nb = N // R
    return pl.pallas_call(
        _comb_body,
        grid=(nb,),
        in_specs=[pl.BlockSpec((R, D), lambda i: (i, 0)),
                  pl.BlockSpec((R, D), lambda i: (i + nb, 0)),
                  pl.BlockSpec((R, CW), lambda i: (i, 0)),
                  pl.BlockSpec((R, CW), lambda i: (i + nb, 0)),
                  pl.BlockSpec((R, D), lambda i: (i, 0)),
                  pl.BlockSpec((D, D), lambda i: (0, 0))],
        out_specs=pl.BlockSpec((R, D), lambda i: (i, 0)),
        out_shape=jax.ShapeDtypeStruct((N, D), jnp.float32),
    )(parts, parts, cnts, cnts, xr, Wl)


def _final_body(p0_ref, p1_ref, c0_ref, c1_ref, xr_ref, w_ref, b_ref,
                wo_ref, bo_ref, o_ref, gsum, gcnt):
    i = pl.program_id(0)

    @pl.when(i == 0)
    def _():
        gsum[...] = jnp.zeros_like(gsum)
        gcnt[...] = jnp.zeros_like(gcnt)

    mean = _mean_of(p0_ref[...], p1_ref[...], c0_ref[...], c1_ref[...])
    h = jnp.maximum(
        jnp.dot(mean, w_ref[...], preferred_element_type=jnp.float32)
        + xr_ref[...], 0.0)
    oh = (b_ref[...] == lax.broadcasted_iota(jnp.int32, (R, G), 1)
          ).astype(jnp.float32)
    gsum[...] += lax.dot_general(oh, h, (((0,), (0,)), ((), ())),
                                 preferred_element_type=jnp.float32)
    gcnt[...] += lax.dot_general(oh, jnp.ones((R, D), jnp.float32),
                                 (((0,), (0,)), ((), ())),
                                 preferred_element_type=jnp.float32)

    @pl.when(i == pl.num_programs(0) - 1)
    def _():
        g = gsum[...] / jnp.maximum(gcnt[...], 1.0)
        o_ref[...] = (jnp.dot(g, wo_ref[...],
                              preferred_element_type=jnp.float32) + bo_ref[...])


def _final(parts, cnts, xr, Wl, batch2d, Wo, bo2d):
    nb = N // R
    return pl.pallas_call(
        _final_body,
        grid=(nb,),
        in_specs=[pl.BlockSpec((R, D), lambda i: (i, 0)),
                  pl.BlockSpec((R, D), lambda i: (i + nb, 0)),
                  pl.BlockSpec((R, CW), lambda i: (i, 0)),
                  pl.BlockSpec((R, CW), lambda i: (i + nb, 0)),
                  pl.BlockSpec((R, D), lambda i: (i, 0)),
                  pl.BlockSpec((D, D), lambda i: (0, 0)),
                  pl.BlockSpec((R, 1), lambda i: (i, 0)),
                  pl.BlockSpec((D, 1), lambda i: (0, 0)),
                  pl.BlockSpec((1, 1), lambda i: (0, 0))],
        out_specs=pl.BlockSpec((G, 1), lambda i: (0, 0)),
        out_shape=jax.ShapeDtypeStruct((G, 1), jnp.float32),
        scratch_shapes=[pltpu.VMEM((G, D), jnp.float32),
                        pltpu.VMEM((G, D), jnp.float32)],
    )(parts, parts, cnts, cnts, xr, Wl, batch2d, Wo, bo2d)


def kernel(x, edge_index, batch, W1l, b1, W1r, W2l, b2, W2r, Wo, bo):
    E = edge_index.shape[1]
    epw = -(-E // (NW * CH * 8)) * CH * 8
    K = epw // CH
    pad = NW * epw - E
    src = jnp.concatenate([edge_index[0], jnp.zeros((pad,), jnp.int32)])
    pad_dst = N + (jnp.arange(pad, dtype=jnp.int32) % (NPAD - N))
    dst = jnp.concatenate([edge_index[1], pad_dst])
    src3 = src.reshape(NW, K, CH)
    dst3 = dst.reshape(NW, K, CH)
    idx3 = jnp.stack([src3, dst3], axis=2).reshape(NW, 2 * K, CH)
    b1_2d = b1.reshape(1, D)
    b2_2d = b2.reshape(1, D)
    bo_2d = bo.reshape(1, 1)
    batch2d = batch.reshape(N, 1)

    cnts = _sc_cnt(dst3)
    parts1 = _sc_agg(x, idx3)
    xr1 = _xw_b(x, W1r, b1_2d)
    h1 = _layer_combine(parts1, cnts, xr1, W1l)

    parts2 = _sc_agg(h1, idx3)
    xr2 = _xw_b(h1, W2r, b2_2d)
    return _final(parts2, cnts, xr2, W2l, batch2d, Wo, bo_2d)
